# baseline (speedup 1.0000x reference)
import functools

import jax
import jax.numpy as jnp
from jax import lax
from jax.experimental import pallas as pl
from jax.experimental.pallas import tpu as pltpu
from jax.experimental.pallas import tpu_sc as plsc

NUM_EXPERTS = 8
TOP_K = 2
D_MODEL = 768
D_FF = 4 * D_MODEL
T = 2048
BLK = 128
NBLK = (TOP_K * T) // BLK + NUM_EXPERTS
S_PAD = NBLK * BLK


def _router_body(x_ref, gw_ref, e_ref, w_ref):
    x = x_ref[...]
    gw = gw_ref[...]
    logits = lax.dot_general(x, gw, (((1,), (1,)), ((), ())),
                             preferred_element_type=jnp.float32)
    m = jnp.max(logits, axis=-1, keepdims=True)
    p = jnp.exp(logits - m)
    p = p / jnp.sum(p, axis=-1, keepdims=True)
    idx = lax.broadcasted_iota(jnp.int32, (T, NUM_EXPERTS), 1)
    m1 = jnp.max(p, axis=-1, keepdims=True)
    e0 = jnp.min(jnp.where(p == m1, idx, NUM_EXPERTS), axis=-1, keepdims=True)
    p2 = jnp.where(idx == e0, -1.0, p)
    m2 = jnp.max(p2, axis=-1, keepdims=True)
    e1 = jnp.min(jnp.where(p2 == m2, idx, NUM_EXPERTS), axis=-1, keepdims=True)
    s = m1 + m2
    e_ref[...] = jnp.concatenate([e0, e1], axis=1)
    w_ref[...] = jnp.concatenate([m1 / s, m2 / s], axis=1)


def _router(x2, gate_w):
    return pl.pallas_call(
        _router_body,
        out_shape=(jax.ShapeDtypeStruct((T, TOP_K), jnp.int32),
                   jax.ShapeDtypeStruct((T, TOP_K), jnp.float32)),
    )(x2, gate_w)


def _gmm_body(gid_ref, ord_ref, fst_ref, x_ref, w1_hbm, w2_hbm, o_ref,
              w1buf, w2buf, w1bf, w2bf, sems):
    b = pl.program_id(0)
    e = gid_ref[b]
    par = lax.rem(ord_ref[b], 2)
    nxt = fst_ref[b]

    @pl.when(b == 0)
    def _():
        pltpu.make_async_copy(w1_hbm.at[e], w1buf.at[0], sems.at[0, 0]).start()
        pltpu.make_async_copy(w2_hbm.at[e], w2buf.at[0], sems.at[1, 0]).start()

    @pl.when(nxt >= 0)
    def _():
        np_ = lax.rem(ord_ref[b] + 1, 2)
        pltpu.make_async_copy(w1_hbm.at[nxt], w1buf.at[np_],
                              sems.at[0, np_]).start()
        pltpu.make_async_copy(w2_hbm.at[nxt], w2buf.at[np_],
                              sems.at[1, np_]).start()

    @pl.when(nxt >= -1)
    def _():
        pltpu.make_async_copy(w1_hbm.at[e], w1buf.at[par],
                              sems.at[0, par]).wait()
        pltpu.make_async_copy(w2_hbm.at[e], w2buf.at[par],
                              sems.at[1, par]).wait()
        w1bf[...] = w1buf[par].astype(jnp.bfloat16)
        w2bf[...] = w2buf[par].astype(jnp.bfloat16)

    xb = x_ref[...].astype(jnp.bfloat16)
    h = lax.dot_general(xb, w1bf[...], (((1,), (1,)), ((), ())),
                        preferred_element_type=jnp.float32)
    h = 0.5 * h * (1.0 + lax.erf(h * 0.7071067811865476))
    o_ref[...] = lax.dot_general(h.astype(jnp.bfloat16), w2bf[...],
                                 (((1,), (1,)), ((), ())),
                                 preferred_element_type=jnp.float32)


def _gmm(xg, W1, W2, gid, ordv, fst):
    grid_spec = pltpu.PrefetchScalarGridSpec(
        num_scalar_prefetch=3,
        grid=(NBLK,),
        in_specs=[
            pl.BlockSpec((BLK, D_MODEL), lambda b, *_: (b, 0)),
            pl.BlockSpec(memory_space=pltpu.MemorySpace.HBM),
            pl.BlockSpec(memory_space=pltpu.MemorySpace.HBM),
        ],
        out_specs=pl.BlockSpec((BLK, D_MODEL), lambda b, *_: (b, 0)),
        scratch_shapes=[
            pltpu.VMEM((2, D_FF, D_MODEL), jnp.float32),
            pltpu.VMEM((2, D_MODEL, D_FF), jnp.float32),
            pltpu.VMEM((D_FF, D_MODEL), jnp.bfloat16),
            pltpu.VMEM((D_MODEL, D_FF), jnp.bfloat16),
            pltpu.SemaphoreType.DMA((2, 2)),
        ],
    )
    return pl.pallas_call(
        _gmm_body,
        grid_spec=grid_spec,
        out_shape=jax.ShapeDtypeStruct((S_PAD, D_MODEL), jnp.float32),
        compiler_params=pltpu.CompilerParams(
            vmem_limit_bytes=100 * 1024 * 1024),
    )(gid, ordv, fst, xg, W1, W2)


_SC_MESH = dict(core_axis_name="c", subcore_axis_name="s",
                num_cores=2, num_subcores=16)
NW = 32
GID_PAD = 48


def _dispatch_sc(experts_flat):
    @functools.partial(
        pl.kernel,
        out_type=(jax.ShapeDtypeStruct((S_PAD,), jnp.int32),
                  jax.ShapeDtypeStruct((2 * T,), jnp.int32),
                  jax.ShapeDtypeStruct((GID_PAD,), jnp.int32),
                  jax.ShapeDtypeStruct((GID_PAD,), jnp.int32),
                  jax.ShapeDtypeStruct((GID_PAD,), jnp.int32)),
        mesh=plsc.VectorSubcoreMesh(**_SC_MESH),
        scratch_types=[pltpu.VMEM((2 * T,), jnp.int32),
                       pltpu.VMEM((S_PAD,), jnp.int32),
                       pltpu.VMEM((2 * T,), jnp.int32),
                       pltpu.VMEM((GID_PAD,), jnp.int32),
                       pltpu.VMEM((GID_PAD,), jnp.int32),
                       pltpu.VMEM((GID_PAD,), jnp.int32)],
        compiler_params=pltpu.CompilerParams(needs_layout_passes=False),
    )
    def k(ef_hbm, tok_hbm, pos_hbm, gid_hbm, ord_hbm, fst_hbm,
          ev_v, tok_v, pos_v, gid_v, ord_v, fst_v):
        @pl.when((lax.axis_index("c") == 0) & (lax.axis_index("s") == 0))
        def _():
            pltpu.sync_copy(ef_hbm, ev_v)
            lanes = lax.iota(jnp.int32, 16)
            zero = jnp.zeros((16,), jnp.int32)

            def count_body(c, cnt):
                v = ev_v[pl.ds(c * 16, 16)]
                return tuple(cnt[e] + jnp.sum(jnp.where(v == e, 1, 0))
                             for e in range(NUM_EXPERTS))

            cnt = lax.fori_loop(0, 2 * T // 16, count_body,
                                (zero,) * NUM_EXPERTS)
            off, end = [], []
            run = zero
            for e in range(NUM_EXPERTS):
                pad = (cnt[e] + (BLK - 1)) & ~(BLK - 1)
                off.append(run)
                run = run + pad
                end.append(run)
            act = [end[e] > off[e] for e in range(NUM_EXPERTS)]
            na = zero
            for e in range(NUM_EXPERTS):
                na = na + jnp.where(act[e], 1, 0)
            for c in range(GID_PAD // 16):
                bv = (lanes + c * 16) * BLK
                g = zero
                o = zero
                f = zero
                for e in range(NUM_EXPERTS):
                    g = g + jnp.where(bv >= end[e], 1, 0)
                    o = o + jnp.where(act[e] & (end[e] <= bv), 1, 0)
                    f = f + jnp.where(act[e] & (off[e] == bv), 1, 0)
                g = jnp.minimum(g, NUM_EXPERTS - 1)
                nxt = jnp.full((16,), -1, jnp.int32)
                for e in range(NUM_EXPERTS - 1, -1, -1):
                    nxt = jnp.where(act[e] & (g < e), e, nxt)
                gid_v[pl.ds(c * 16, 16)] = g
                ord_v[pl.ds(c * 16, 16)] = jnp.minimum(o, na - 1)
                fst_v[pl.ds(c * 16, 16)] = jnp.where(f > 0, nxt, -2)

            def zero_body(i, carry):
                tok_v[pl.ds(i * 16, 16)] = (lanes + i * 16) & (T - 1)
                return carry

            lax.fori_loop(0, S_PAD // 16, zero_body, 0)

            def scatter_body(c, cnt2):
                a = lanes + c * 16
                v = ev_v[pl.ds(c * 16, 16)]
                t = a >> 1
                posc = zero
                cnt2 = list(cnt2)
                for e in range(NUM_EXPERTS):
                    m = v == e
                    r = jnp.cumsum(jnp.where(m, 1, 0))
                    posv = off[e] + cnt2[e] + r - 1
                    plsc.store_scatter(tok_v, [posv], t, mask=m)
                    posc = jnp.where(m, posv, posc)
                    cnt2[e] = cnt2[e] + jnp.sum(jnp.where(m, 1, 0))
                idxp = ((a & 1) << 11) + t
                plsc.store_scatter(pos_v, [idxp], posc)
                return tuple(cnt2)

            lax.fori_loop(0, 2 * T // 16, scatter_body, (zero,) * NUM_EXPERTS)
            pltpu.sync_copy(tok_v, tok_hbm)
            pltpu.sync_copy(pos_v, pos_hbm)
            pltpu.sync_copy(gid_v, gid_hbm)
            pltpu.sync_copy(ord_v, ord_hbm)
            pltpu.sync_copy(fst_v, fst_hbm)

    return k(experts_flat)


_GCH = 4


def _gather_x(x2, tok_slot):
    b_per_w = S_PAD // NW
    ch = b_per_w // _GCH

    @functools.partial(
        pl.kernel,
        out_type=jax.ShapeDtypeStruct((S_PAD, D_MODEL), jnp.float32),
        mesh=plsc.VectorSubcoreMesh(**_SC_MESH),
        scratch_types=[pltpu.VMEM((b_per_w,), jnp.int32),
                       pltpu.VMEM((_GCH, ch, D_MODEL), jnp.float32),
                       pltpu.SemaphoreType.DMA,
                       pltpu.SemaphoreType.DMA],
        compiler_params=pltpu.CompilerParams(needs_layout_passes=False),
    )
    def k(x_hbm, tok_hbm, out_hbm, idx_v, rows_v, gsem, ssem):
        wid = lax.axis_index("s") * 2 + lax.axis_index("c")
        base = wid * b_per_w
        pltpu.sync_copy(tok_hbm.at[pl.ds(base, b_per_w)], idx_v)
        gcp = [pltpu.async_copy(x_hbm.at[idx_v.at[pl.ds(g * ch, ch)]],
                                rows_v.at[g], gsem)
               for g in range(_GCH)]
        scp = []
        for g in range(_GCH):
            gcp[g].wait()
            scp.append(pltpu.async_copy(
                rows_v.at[g], out_hbm.at[pl.ds(base + g * ch, ch)], ssem))
        for c in scp:
            c.wait()

    return k(x2, tok_slot)


def _combine_sc(yg, pos, wts_flat):
    tpw = T // NW
    half = tpw // 2

    @functools.partial(
        pl.kernel,
        out_type=jax.ShapeDtypeStruct((T, D_MODEL), jnp.float32),
        mesh=plsc.VectorSubcoreMesh(**_SC_MESH),
        scratch_types=[pltpu.VMEM((half,), jnp.int32),
                       pltpu.VMEM((half,), jnp.int32),
                       pltpu.VMEM((half, D_MODEL), jnp.float32),
                       pltpu.VMEM((half, D_MODEL), jnp.float32),
                       pltpu.VMEM((2 * half,), jnp.float32),
                       pltpu.VMEM((half, D_MODEL), jnp.float32),
                       pltpu.SemaphoreType.DMA],
        compiler_params=pltpu.CompilerParams(needs_layout_passes=False),
    )
    def k(yg_hbm, pos_hbm, w_hbm, out_hbm, i0_v, i1_v, y0_v, y1_v, w_v, o_v,
          sem):
        wid = lax.axis_index("s") * 2 + lax.axis_index("c")
        for h in range(2):
            tbase = wid * tpw + h * half
            pltpu.sync_copy(pos_hbm.at[pl.ds(tbase, half)], i0_v)
            pltpu.sync_copy(pos_hbm.at[pl.ds(T + tbase, half)], i1_v)
            c0 = pltpu.async_copy(yg_hbm.at[i0_v], y0_v, sem)
            c1 = pltpu.async_copy(yg_hbm.at[i1_v], y1_v, sem)
            pltpu.sync_copy(w_hbm.at[pl.ds(2 * tbase, 2 * half)], w_v)
            c0.wait()
            c1.wait()

            def tok_body(i, carry):
                w0 = plsc.load_gather(w_v, [jnp.full((16,), 2 * i, jnp.int32)])
                w1 = plsc.load_gather(w_v,
                                      [jnp.full((16,), 2 * i + 1, jnp.int32)])
                for j in range(D_MODEL // 16):
                    sl = pl.ds(j * 16, 16)
                    o_v[i, sl] = w0 * y0_v[i, sl] + w1 * y1_v[i, sl]
                return carry

            lax.fori_loop(0, half, tok_body, 0)
            pltpu.sync_copy(o_v, out_hbm.at[pl.ds(tbase, half)])

    return k(yg, pos, wts_flat)


def kernel(x, gate_w, W1, W2):
    x2 = x.reshape(T, D_MODEL)
    experts, wts = _router(x2, gate_w)
    tok_slot, pos, gid, ordv, fst = _dispatch_sc(experts.reshape(-1))
    xg = _gather_x(x2, tok_slot)
    yg = _gmm(xg, W1, W2, gid, ordv, fst)
    out = _combine_sc(yg, pos, wts.reshape(-1))
    return out.reshape(x.shape)


# speedup vs baseline: 1.0413x; 1.0413x over previous
import functools

import jax
import jax.numpy as jnp
from jax import lax
from jax.experimental import pallas as pl
from jax.experimental.pallas import tpu as pltpu
from jax.experimental.pallas import tpu_sc as plsc

NUM_EXPERTS = 8
TOP_K = 2
D_MODEL = 768
D_FF = 4 * D_MODEL
T = 2048
BLK = 128
NBLK = (TOP_K * T) // BLK + NUM_EXPERTS
S_PAD = NBLK * BLK


def _router_body(x_ref, gw_ref, e_ref, w_ref):
    x = x_ref[...]
    gw = gw_ref[...]
    logits = lax.dot_general(x, gw, (((1,), (1,)), ((), ())),
                             preferred_element_type=jnp.float32)
    m = jnp.max(logits, axis=-1, keepdims=True)
    p = jnp.exp(logits - m)
    p = p / jnp.sum(p, axis=-1, keepdims=True)
    idx = lax.broadcasted_iota(jnp.int32, (T, NUM_EXPERTS), 1)
    m1 = jnp.max(p, axis=-1, keepdims=True)
    e0 = jnp.min(jnp.where(p == m1, idx, NUM_EXPERTS), axis=-1, keepdims=True)
    p2 = jnp.where(idx == e0, -1.0, p)
    m2 = jnp.max(p2, axis=-1, keepdims=True)
    e1 = jnp.min(jnp.where(p2 == m2, idx, NUM_EXPERTS), axis=-1, keepdims=True)
    s = m1 + m2
    e_ref[...] = jnp.concatenate([e0, e1], axis=1)
    w_ref[...] = jnp.concatenate([m1 / s, m2 / s], axis=1)


def _router(x2, gate_w):
    return pl.pallas_call(
        _router_body,
        out_shape=(jax.ShapeDtypeStruct((T, TOP_K), jnp.int32),
                   jax.ShapeDtypeStruct((T, TOP_K), jnp.float32)),
    )(x2, gate_w)


def _gmm_body(gid_ref, ord_ref, fst_ref, x_ref, w1_hbm, w2_hbm, o_ref,
              w1buf, w2buf, sems):
    b = pl.program_id(0)
    e = gid_ref[b]
    par = lax.rem(ord_ref[b], 2)
    nxt = fst_ref[b]

    @pl.when(b == 0)
    def _():
        pltpu.make_async_copy(w1_hbm.at[e], w1buf.at[0], sems.at[0, 0]).start()
        pltpu.make_async_copy(w2_hbm.at[e], w2buf.at[0], sems.at[1, 0]).start()

    @pl.when(nxt >= 0)
    def _():
        np_ = lax.rem(ord_ref[b] + 1, 2)
        pltpu.make_async_copy(w1_hbm.at[nxt], w1buf.at[np_],
                              sems.at[0, np_]).start()
        pltpu.make_async_copy(w2_hbm.at[nxt], w2buf.at[np_],
                              sems.at[1, np_]).start()

    @pl.when(nxt >= -1)
    def _():
        pltpu.make_async_copy(w1_hbm.at[e], w1buf.at[par],
                              sems.at[0, par]).wait()
        pltpu.make_async_copy(w2_hbm.at[e], w2buf.at[par],
                              sems.at[1, par]).wait()

    xb = x_ref[...]
    h = lax.dot_general(xb, w1buf[par], (((1,), (1,)), ((), ())),
                        preferred_element_type=jnp.float32)
    h = 0.5 * h * (1.0 + lax.erf(h * 0.7071067811865476))
    o_ref[...] = lax.dot_general(h, w2buf[par], (((1,), (1,)), ((), ())),
                                 preferred_element_type=jnp.float32)


def _gmm(xg, W1, W2, gid, ordv, fst):
    grid_spec = pltpu.PrefetchScalarGridSpec(
        num_scalar_prefetch=3,
        grid=(NBLK,),
        in_specs=[
            pl.BlockSpec((BLK, D_MODEL), lambda b, *_: (b, 0)),
            pl.BlockSpec(memory_space=pltpu.MemorySpace.HBM),
            pl.BlockSpec(memory_space=pltpu.MemorySpace.HBM),
        ],
        out_specs=pl.BlockSpec((BLK, D_MODEL), lambda b, *_: (b, 0)),
        scratch_shapes=[
            pltpu.VMEM((2, D_FF, D_MODEL), jnp.float32),
            pltpu.VMEM((2, D_MODEL, D_FF), jnp.float32),
            pltpu.SemaphoreType.DMA((2, 2)),
        ],
    )
    return pl.pallas_call(
        _gmm_body,
        grid_spec=grid_spec,
        out_shape=jax.ShapeDtypeStruct((S_PAD, D_MODEL), jnp.float32),
        compiler_params=pltpu.CompilerParams(
            vmem_limit_bytes=100 * 1024 * 1024),
    )(gid, ordv, fst, xg, W1, W2)


_SC_MESH = dict(core_axis_name="c", subcore_axis_name="s",
                num_cores=2, num_subcores=16)
NW = 32
GID_PAD = 48


def _dispatch_sc(experts_flat):
    @functools.partial(
        pl.kernel,
        out_type=(jax.ShapeDtypeStruct((S_PAD,), jnp.int32),
                  jax.ShapeDtypeStruct((2 * T,), jnp.int32),
                  jax.ShapeDtypeStruct((GID_PAD,), jnp.int32),
                  jax.ShapeDtypeStruct((GID_PAD,), jnp.int32),
                  jax.ShapeDtypeStruct((GID_PAD,), jnp.int32)),
        mesh=plsc.VectorSubcoreMesh(**_SC_MESH),
        scratch_types=[pltpu.VMEM((2 * T,), jnp.int32),
                       pltpu.VMEM((S_PAD,), jnp.int32),
                       pltpu.VMEM((2 * T,), jnp.int32),
                       pltpu.VMEM((GID_PAD,), jnp.int32),
                       pltpu.VMEM((GID_PAD,), jnp.int32),
                       pltpu.VMEM((GID_PAD,), jnp.int32)],
        compiler_params=pltpu.CompilerParams(needs_layout_passes=False),
    )
    def k(ef_hbm, tok_hbm, pos_hbm, gid_hbm, ord_hbm, fst_hbm,
          ev_v, tok_v, pos_v, gid_v, ord_v, fst_v):
        @pl.when((lax.axis_index("c") == 0) & (lax.axis_index("s") == 0))
        def _():
            pltpu.sync_copy(ef_hbm, ev_v)
            lanes = lax.iota(jnp.int32, 16)
            zero = jnp.zeros((16,), jnp.int32)

            def count_body(c, cnt):
                v = ev_v[pl.ds(c * 16, 16)]
                return tuple(cnt[e] + jnp.sum(jnp.where(v == e, 1, 0))
                             for e in range(NUM_EXPERTS))

            cnt = lax.fori_loop(0, 2 * T // 16, count_body,
                                (zero,) * NUM_EXPERTS)
            off, end = [], []
            run = zero
            for e in range(NUM_EXPERTS):
                pad = (cnt[e] + (BLK - 1)) & ~(BLK - 1)
                off.append(run)
                run = run + pad
                end.append(run)
            act = [end[e] > off[e] for e in range(NUM_EXPERTS)]
            na = zero
            for e in range(NUM_EXPERTS):
                na = na + jnp.where(act[e], 1, 0)
            for c in range(GID_PAD // 16):
                bv = (lanes + c * 16) * BLK
                g = zero
                o = zero
                f = zero
                for e in range(NUM_EXPERTS):
                    g = g + jnp.where(bv >= end[e], 1, 0)
                    o = o + jnp.where(act[e] & (end[e] <= bv), 1, 0)
                    f = f + jnp.where(act[e] & (off[e] == bv), 1, 0)
                g = jnp.minimum(g, NUM_EXPERTS - 1)
                nxt = jnp.full((16,), -1, jnp.int32)
                for e in range(NUM_EXPERTS - 1, -1, -1):
                    nxt = jnp.where(act[e] & (g < e), e, nxt)
                gid_v[pl.ds(c * 16, 16)] = g
                ord_v[pl.ds(c * 16, 16)] = jnp.minimum(o, na - 1)
                fst_v[pl.ds(c * 16, 16)] = jnp.where(f > 0, nxt, -2)

            def zero_body(i, carry):
                tok_v[pl.ds(i * 16, 16)] = (lanes + i * 16) & (T - 1)
                return carry

            lax.fori_loop(0, S_PAD // 16, zero_body, 0)

            def scatter_body(c, cnt2):
                a = lanes + c * 16
                v = ev_v[pl.ds(c * 16, 16)]
                t = a >> 1
                posc = zero
                cnt2 = list(cnt2)
                for e in range(NUM_EXPERTS):
                    m = v == e
                    r = jnp.cumsum(jnp.where(m, 1, 0))
                    posv = off[e] + cnt2[e] + r - 1
                    plsc.store_scatter(tok_v, [posv], t, mask=m)
                    posc = jnp.where(m, posv, posc)
                    cnt2[e] = cnt2[e] + jnp.sum(jnp.where(m, 1, 0))
                idxp = ((a & 1) << 11) + t
                plsc.store_scatter(pos_v, [idxp], posc)
                return tuple(cnt2)

            lax.fori_loop(0, 2 * T // 16, scatter_body, (zero,) * NUM_EXPERTS)
            pltpu.sync_copy(tok_v, tok_hbm)
            pltpu.sync_copy(pos_v, pos_hbm)
            pltpu.sync_copy(gid_v, gid_hbm)
            pltpu.sync_copy(ord_v, ord_hbm)
            pltpu.sync_copy(fst_v, fst_hbm)

    return k(experts_flat)


_GCH = 4


def _gather_x(x2, tok_slot):
    b_per_w = S_PAD // NW
    ch = b_per_w // _GCH

    @functools.partial(
        pl.kernel,
        out_type=jax.ShapeDtypeStruct((S_PAD, D_MODEL), jnp.float32),
        mesh=plsc.VectorSubcoreMesh(**_SC_MESH),
        scratch_types=[pltpu.VMEM((b_per_w,), jnp.int32),
                       pltpu.VMEM((_GCH, ch, D_MODEL), jnp.float32),
                       pltpu.SemaphoreType.DMA,
                       pltpu.SemaphoreType.DMA],
        compiler_params=pltpu.CompilerParams(needs_layout_passes=False),
    )
    def k(x_hbm, tok_hbm, out_hbm, idx_v, rows_v, gsem, ssem):
        wid = lax.axis_index("s") * 2 + lax.axis_index("c")
        base = wid * b_per_w
        pltpu.sync_copy(tok_hbm.at[pl.ds(base, b_per_w)], idx_v)
        gcp = [pltpu.async_copy(x_hbm.at[idx_v.at[pl.ds(g * ch, ch)]],
                                rows_v.at[g], gsem)
               for g in range(_GCH)]
        scp = []
        for g in range(_GCH):
            gcp[g].wait()
            scp.append(pltpu.async_copy(
                rows_v.at[g], out_hbm.at[pl.ds(base + g * ch, ch)], ssem))
        for c in scp:
            c.wait()

    return k(x2, tok_slot)


def _combine_sc(yg, pos, wts_flat):
    tpw = T // NW
    NQ = 4
    q16 = tpw // NQ

    @functools.partial(
        pl.kernel,
        out_type=jax.ShapeDtypeStruct((T, D_MODEL), jnp.float32),
        mesh=plsc.VectorSubcoreMesh(**_SC_MESH),
        scratch_types=[pltpu.VMEM((tpw,), jnp.int32),
                       pltpu.VMEM((tpw,), jnp.int32),
                       pltpu.VMEM((2, q16, D_MODEL), jnp.float32),
                       pltpu.VMEM((2, q16, D_MODEL), jnp.float32),
                       pltpu.VMEM((2 * tpw,), jnp.float32),
                       pltpu.VMEM((2, q16, D_MODEL), jnp.float32),
                       pltpu.SemaphoreType.DMA,
                       pltpu.SemaphoreType.DMA],
        compiler_params=pltpu.CompilerParams(needs_layout_passes=False),
    )
    def k(yg_hbm, pos_hbm, w_hbm, out_hbm, i0_v, i1_v, y0_v, y1_v, w_v, o_v,
          gsem, ssem):
        wid = lax.axis_index("s") * 2 + lax.axis_index("c")
        tbase = wid * tpw
        pltpu.sync_copy(pos_hbm.at[pl.ds(tbase, tpw)], i0_v)
        pltpu.sync_copy(pos_hbm.at[pl.ds(T + tbase, tpw)], i1_v)
        pltpu.sync_copy(w_hbm.at[pl.ds(2 * tbase, 2 * tpw)], w_v)

        def gathers(q):
            return (pltpu.async_copy(yg_hbm.at[i0_v.at[pl.ds(q * q16, q16)]],
                                     y0_v.at[q % 2], gsem),
                    pltpu.async_copy(yg_hbm.at[i1_v.at[pl.ds(q * q16, q16)]],
                                     y1_v.at[q % 2], gsem))

        gcp = {0: gathers(0)}
        scp = {}
        for q in range(NQ):
            if q + 1 < NQ:
                gcp[q + 1] = gathers(q + 1)
            if q >= 2:
                scp[q - 2].wait()
            gcp[q][0].wait()
            gcp[q][1].wait()
            p = q % 2

            def tok_body(i, carry):
                w0 = plsc.load_gather(
                    w_v, [jnp.full((16,), 2 * (q * q16) + 2 * i, jnp.int32)])
                w1 = plsc.load_gather(
                    w_v, [jnp.full((16,), 2 * (q * q16) + 2 * i + 1,
                                   jnp.int32)])
                for j in range(D_MODEL // 16):
                    sl = pl.ds(j * 16, 16)
                    o_v[p, i, sl] = w0 * y0_v[p, i, sl] + w1 * y1_v[p, i, sl]
                return carry

            lax.fori_loop(0, q16, tok_body, 0)
            scp[q] = pltpu.async_copy(
                o_v.at[p], out_hbm.at[pl.ds(tbase + q * q16, q16)], ssem)
        scp[NQ - 2].wait()
        scp[NQ - 1].wait()

    return k(yg, pos, wts_flat)


def kernel(x, gate_w, W1, W2):
    x2 = x.reshape(T, D_MODEL)
    experts, wts = _router(x2, gate_w)
    tok_slot, pos, gid, ordv, fst = _dispatch_sc(experts.reshape(-1))
    xg = _gather_x(x2, tok_slot)
    yg = _gmm(xg, W1, W2, gid, ordv, fst)
    out = _combine_sc(yg, pos, wts.reshape(-1))
    return out.reshape(x.shape)
